# baseline (speedup 1.0000x reference)
import functools

import jax
import jax.numpy as jnp
from jax import lax
from jax.experimental import pallas as pl
from jax.experimental.pallas import tpu as pltpu
from jax.experimental.pallas import tpu_sc as plsc

B = 4
S = 2048
D = 2048
NR = 16
NVT = 100
TOPK = 2
BS = 512
NS = S // BS

NC = 2
NSC = 16
NW = NC * NSC
WPB = NW // B


def _route_body(x_ref, m_ref, w_ref, wout_ref, oout_ref, acc_ref, cnt_ref):
    ns = pl.program_id(1)
    x = x_ref[0]
    m = m_ref[0]
    part = lax.dot_general(m, x, (((1,), (0,)), ((), ())),
                           preferred_element_type=jnp.float32,
                           precision=lax.Precision.HIGHEST)
    pc = jnp.sum(m)

    @pl.when(ns == 0)
    def _():
        acc_ref[...] = part
        cnt_ref[0] = pc

    @pl.when(ns != 0)
    def _():
        acc_ref[...] = acc_ref[...] + part
        cnt_ref[0] = cnt_ref[0] + pc

    @pl.when(ns == NS - 1)
    def _():
        cnt = jnp.maximum(cnt_ref[0], 1.0)
        sent = acc_ref[...] / cnt
        logits = lax.dot_general(sent, w_ref[...], (((1,), (1,)), ((), ())),
                                 preferred_element_type=jnp.float32,
                                 precision=lax.Precision.HIGHEST)
        mx = jnp.max(logits, axis=1, keepdims=True)
        e = jnp.exp(logits - mx)
        p = e / jnp.sum(e, axis=1, keepdims=True)
        iota = lax.broadcasted_iota(jnp.int32, (1, NR), 1)
        m1 = jnp.max(p, axis=1, keepdims=True)
        i1 = jnp.min(jnp.where(p == m1, iota, NR), axis=1, keepdims=True)
        p2 = jnp.where(iota == i1, -1.0, p)
        m2 = jnp.max(p2, axis=1, keepdims=True)
        i2 = jnp.min(jnp.where(p2 == m2, iota, NR), axis=1, keepdims=True)
        wout_ref[...] = jnp.where(iota == 0, m1,
                                  jnp.where(iota == 1, m2, 0.0)).reshape(1, 1, NR)
        oout_ref[...] = jnp.where(iota == 0, i1 * NVT,
                                  jnp.where(iota == 1, i2 * NVT, 0)).reshape(1, 1, NR)


def _route(inputs_embeds, mask3, W_router):
    return pl.pallas_call(
        _route_body,
        grid=(B, NS),
        in_specs=[
            pl.BlockSpec((1, BS, D), lambda b, ns: (b, ns, 0)),
            pl.BlockSpec((1, 1, BS), lambda b, ns: (b, 0, ns)),
            pl.BlockSpec((NR, D), lambda b, ns: (0, 0)),
        ],
        out_specs=[
            pl.BlockSpec((1, 1, NR), lambda b, ns: (b, 0, 0)),
            pl.BlockSpec((1, 1, NR), lambda b, ns: (b, 0, 0)),
        ],
        out_shape=[
            jax.ShapeDtypeStruct((B, 1, NR), jnp.float32),
            jax.ShapeDtypeStruct((B, 1, NR), jnp.int32),
        ],
        scratch_shapes=[
            pltpu.VMEM((1, D), jnp.float32),
            pltpu.SMEM((1,), jnp.float32),
        ],
    )(inputs_embeds, mask3, W_router)


def _combine_body(w_hbm, off_hbm, emb_hbm, out_hbm,
                  w_v, off_v, r0_v, r1_v, o_v, sem0, sem1):
    cid = lax.axis_index("c")
    sid = lax.axis_index("s")
    wid = sid * NC + cid
    b = wid // WPB
    lc = wid % WPB
    start_r = lc * 12 + jnp.minimum(lc, 4)

    pltpu.sync_copy(w_hbm, w_v)
    pltpu.sync_copy(off_hbm, off_v)

    wrow = w_v[b, :]
    orow = off_v[b, :]
    w0 = jnp.full((16,), wrow[0], jnp.float32)
    w1 = jnp.full((16,), wrow[1], jnp.float32)
    o0 = orow[0]
    o1 = orow[1]

    src0 = (o0 + start_r) * D
    src1 = (o1 + start_r) * D
    dst = (b * NVT + start_r) * D

    def work(nrows):
        n = nrows * D
        cp0 = pltpu.async_copy(emb_hbm.at[pl.ds(src0, n)], r0_v.at[pl.ds(0, n)], sem0)
        cp1 = pltpu.async_copy(emb_hbm.at[pl.ds(src1, n)], r1_v.at[pl.ds(0, n)], sem1)
        cp0.wait()
        cp1.wait()

        def body(c, carry):
            base = c * 64
            for u in range(4):
                sl = pl.ds(base + u * 16, 16)
                o_v[sl] = r0_v[sl] * w0 + r1_v[sl] * w1
            return carry

        lax.fori_loop(0, n // 64, body, 0)
        pltpu.sync_copy(o_v.at[pl.ds(0, n)], out_hbm.at[pl.ds(dst, n)])

    @pl.when(lc < 4)
    def _():
        work(13)

    @pl.when(lc >= 4)
    def _():
        work(12)


@functools.lru_cache(maxsize=1)
def _combine():
    return pl.kernel(
        _combine_body,
        mesh=plsc.VectorSubcoreMesh(core_axis_name="c", subcore_axis_name="s"),
        out_type=jax.ShapeDtypeStruct((B * NVT * D,), jnp.float32),
        scratch_types=[
            pltpu.VMEM((B, NR), jnp.float32),
            pltpu.VMEM((B, NR), jnp.int32),
            pltpu.VMEM((13 * D,), jnp.float32),
            pltpu.VMEM((13 * D,), jnp.float32),
            pltpu.VMEM((13 * D,), jnp.float32),
            pltpu.SemaphoreType.DMA,
            pltpu.SemaphoreType.DMA,
        ],
    )


def kernel(indices, input_ids, inputs_embeds, attention_mask, embedding, W_router):
    mask3 = attention_mask.astype(jnp.float32).reshape(B, 1, S)
    w_pad, off_pad = _route(inputs_embeds, mask3, W_router)
    out = _combine()(w_pad.reshape(B, NR), off_pad.reshape(B, NR),
                     embedding.reshape(embedding.size))
    return out.reshape(B, NVT, D)
